# Initial kernel scaffold; baseline (speedup 1.0000x reference)
import jax
import jax.numpy as jnp
from jax.experimental import pallas as pl


def kernel(embeddings, component_idx, W, b):
    raise NotImplementedError("write your pallas kernel here")


# speedup vs baseline: 8.2009x; 8.2009x over previous
import functools

import jax
import jax.numpy as jnp
from jax import lax
from jax.experimental import pallas as pl

NCOMP = 16
NCLASS = 2
D = 768
TILE = 2048


def _mm_select_kernel(cid_ref, emb_ref, w_ref, b_ref, out_ref):
    logits = jnp.dot(emb_ref[...], w_ref[...], preferred_element_type=jnp.float32)
    logits = logits + b_ref[...]
    lane = lax.broadcasted_iota(jnp.int32, (TILE, NCOMP * NCLASS), 1)
    sel = (lane // NCLASS) == cid_ref[...]
    masked = jnp.where(sel, logits, 0.0)
    even = (lane % NCLASS) == 0
    out0 = jnp.sum(jnp.where(even, masked, 0.0), axis=1, keepdims=True)
    out1 = jnp.sum(jnp.where(even, 0.0, masked), axis=1, keepdims=True)
    out_ref[...] = jnp.concatenate([out0, out1], axis=1)


def kernel(embeddings, component_idx, W, b):
    B = embeddings.shape[0]
    cid = component_idx.astype(jnp.int32).reshape(B, 1)
    w_full = jnp.transpose(W, (1, 0, 2)).reshape(D, NCOMP * NCLASS)
    b_full = b.reshape(1, NCOMP * NCLASS)

    out = pl.pallas_call(
        _mm_select_kernel,
        grid=(B // TILE,),
        in_specs=[
            pl.BlockSpec((TILE, 1), lambda i: (i, 0)),
            pl.BlockSpec((TILE, D), lambda i: (i, 0)),
            pl.BlockSpec((D, NCOMP * NCLASS), lambda i: (0, 0)),
            pl.BlockSpec((1, NCOMP * NCLASS), lambda i: (0, 0)),
        ],
        out_specs=pl.BlockSpec((TILE, NCLASS), lambda i: (i, 0)),
        out_shape=jax.ShapeDtypeStruct((B, NCLASS), jnp.float32),
    )(cid, embeddings, w_full, b_full)
    return out
